# baseline (speedup 1.0000x reference)
import functools

import jax
import jax.numpy as jnp
from jax import lax
from jax.experimental import pallas as pl
from jax.experimental.pallas import tpu as pltpu
from jax.experimental.pallas import tpu_sc as plsc

_B = 4
_NC = 4096
_NF = 16384
_NPB = _NC // _B
_MPB = _NF // _B
_CIN = 96
_COUT = 48
_MT = 1024
_CT = 512

_NW = 32
_PPW = _NF // _NW
_GP = 64
_NG = _PPW // _GP


def _f2_body(x_ref, g_ref, b_ref, w_ref, bias_ref, o_ref):
    x = x_ref[...]
    mu = jnp.mean(x, axis=1, keepdims=True)
    xc = x - mu
    var = jnp.mean(xc * xc, axis=1, keepdims=True)
    y = xc / jnp.sqrt(var + 1e-5) * g_ref[...] + b_ref[...]
    f2 = (jnp.dot(y, w_ref[...], preferred_element_type=jnp.float32,
                  precision=lax.Precision.HIGHEST)
          + bias_ref[...])
    o_ref[...] = jnp.concatenate(
        [f2, jnp.zeros((_CT, 128 - _COUT), jnp.float32)], axis=1)


def _f2_call(feats, g, b, w, bias):
    return pl.pallas_call(
        _f2_body,
        grid=(_NC // _CT,),
        in_specs=[
            pl.BlockSpec((_CT, _CIN), lambda i: (i, 0)),
            pl.BlockSpec((1, _CIN), lambda i: (0, 0)),
            pl.BlockSpec((1, _CIN), lambda i: (0, 0)),
            pl.BlockSpec((_CIN, _COUT), lambda i: (0, 0)),
            pl.BlockSpec((1, _COUT), lambda i: (0, 0)),
        ],
        out_specs=pl.BlockSpec((_CT, 128), lambda i: (i, 0)),
        out_shape=jax.ShapeDtypeStruct((_NC, 128), jnp.float32),
    )(feats, g, b, w, bias)


def _knn_body(sxyz_ref, cxyz_ref, sf_ref, g_ref, bln_ref, w1_ref, b1_ref,
              i0_ref, i1_ref, i2_ref, w0_ref, w1o_ref, w2_ref, skip_ref,
              b0=0):
    b = b0 + pl.program_id(0)
    fx = sxyz_ref[0, 0:1, :]
    fy = sxyz_ref[0, 1:2, :]
    fz = sxyz_ref[0, 2:3, :]
    cx = cxyz_ref[:, 0:1]
    cy = cxyz_ref[:, 1:2]
    cz = cxyz_ref[:, 2:3]
    dx = cx - fx
    dy = cy - fy
    dz = cz - fz
    d2 = dx * dx + dy * dy + dz * dz
    iotaf = lax.broadcasted_iota(jnp.int32, (_NPB, _MT), 0).astype(jnp.float32)

    idxs = []
    ws = []
    for k in range(3):
        minv = jnp.min(d2, axis=0, keepdims=True)
        cand = jnp.where(d2 == minv, iotaf, jnp.float32(_NPB))
        aminf = jnp.min(cand, axis=0, keepdims=True)
        if k < 2:
            d2 = jnp.where(iotaf == aminf, jnp.float32(jnp.inf), d2)
        dist = jnp.sqrt(jnp.maximum(minv, 1e-12))
        ws.append(1.0 / (dist + 1e-8))
        idxs.append(aminf)
    wsum = ws[0] + ws[1] + ws[2]
    ws = [w / wsum for w in ws]

    gidx = [i.astype(jnp.int32) + b * _NPB for i in idxs]
    i0_ref[...] = gidx[0].reshape(_MT)
    i1_ref[...] = gidx[1].reshape(_MT)
    i2_ref[...] = gidx[2].reshape(_MT)
    w0_ref[...] = ws[0].reshape(_MT)
    w1o_ref[...] = ws[1].reshape(_MT)
    w2_ref[...] = ws[2].reshape(_MT)

    x = sf_ref[...]
    mu = jnp.mean(x, axis=1, keepdims=True)
    xc = x - mu
    var = jnp.mean(xc * xc, axis=1, keepdims=True)
    y = xc / jnp.sqrt(var + 1e-5) * g_ref[...] + bln_ref[...]
    sk = (jnp.dot(y, w1_ref[...], preferred_element_type=jnp.float32,
                  precision=lax.Precision.HIGHEST)
          + b1_ref[...])
    skip_ref[...] = jnp.concatenate(
        [sk, jnp.zeros((_MT, 128 - _COUT), jnp.float32)], axis=1)


def _knn_call(sxyzT, cxyz8, sfeat, g, bln, w1, b1, b0, nb):
    nt = _MPB // _MT
    nh = nb * _MPB
    body = functools.partial(_knn_body, b0=b0)
    return pl.pallas_call(
        body,
        grid=(nb, nt),
        in_specs=[
            pl.BlockSpec((1, 8, _MT), lambda b, t: (0, 0, (b0 + b) * nt + t)),
            pl.BlockSpec((_NPB, 8), lambda b, t: (b0 + b, 0)),
            pl.BlockSpec((_MT, _COUT), lambda b, t: ((b0 + b) * nt + t, 0)),
            pl.BlockSpec((1, _COUT), lambda b, t: (0, 0)),
            pl.BlockSpec((1, _COUT), lambda b, t: (0, 0)),
            pl.BlockSpec((_COUT, _COUT), lambda b, t: (0, 0)),
            pl.BlockSpec((1, _COUT), lambda b, t: (0, 0)),
        ],
        out_specs=[
            pl.BlockSpec((_MT,), lambda b, t: (b * nt + t,)),
            pl.BlockSpec((_MT,), lambda b, t: (b * nt + t,)),
            pl.BlockSpec((_MT,), lambda b, t: (b * nt + t,)),
            pl.BlockSpec((_MT,), lambda b, t: (b * nt + t,)),
            pl.BlockSpec((_MT,), lambda b, t: (b * nt + t,)),
            pl.BlockSpec((_MT,), lambda b, t: (b * nt + t,)),
            pl.BlockSpec((_MT, 128), lambda b, t: (b * nt + t, 0)),
        ],
        out_shape=[
            jax.ShapeDtypeStruct((nh,), jnp.int32),
            jax.ShapeDtypeStruct((nh,), jnp.int32),
            jax.ShapeDtypeStruct((nh,), jnp.int32),
            jax.ShapeDtypeStruct((nh,), jnp.float32),
            jax.ShapeDtypeStruct((nh,), jnp.float32),
            jax.ShapeDtypeStruct((nh,), jnp.float32),
            jax.ShapeDtypeStruct((nh, 128), jnp.float32),
        ],
    )(sxyzT, cxyz8, sfeat, g, bln, w1, b1)


def _sc_combine(f2, idx3, w3, skip, npts):
    ppw = npts // _NW
    ng = ppw // _GP
    mesh = plsc.VectorSubcoreMesh(core_axis_name="c", subcore_axis_name="s")

    @functools.partial(
        pl.kernel,
        out_type=jax.ShapeDtypeStruct((npts * _COUT,), jnp.float32),
        mesh=mesh,
        scratch_types=[
            pltpu.VMEM((ppw,), jnp.int32),
            pltpu.VMEM((ppw,), jnp.int32),
            pltpu.VMEM((ppw,), jnp.int32),
            pltpu.VMEM((ppw,), jnp.float32),
            pltpu.VMEM((ppw,), jnp.float32),
            pltpu.VMEM((ppw,), jnp.float32),
            pltpu.VMEM((ppw,), jnp.int32),
            pltpu.VMEM((ppw * _COUT,), jnp.float32),
            pltpu.VMEM((_GP, 128), jnp.float32),
            pltpu.VMEM((_GP, 128), jnp.float32),
            pltpu.VMEM((_GP, 128), jnp.float32),
            pltpu.VMEM((_GP, 128), jnp.float32),
            pltpu.VMEM((_GP, 128), jnp.float32),
            pltpu.VMEM((_GP, 128), jnp.float32),
            pltpu.VMEM((_GP, 128), jnp.float32),
            pltpu.VMEM((_GP, 128), jnp.float32),
            pltpu.SemaphoreType.DMA,
            pltpu.SemaphoreType.DMA,
        ],
        compiler_params=pltpu.CompilerParams(needs_layout_passes=False),
    )
    def k(f2_hbm, i0_hbm, i1_hbm, i2_hbm, w0_hbm, w1_hbm, w2_hbm,
          skip_hbm, out_hbm,
          i0, i1, i2, w0v, w1v, w2v, ident, out_v,
          r0a, r1a, r2a, ska, r0b, r1b, r2b, skb, semA, semB):
        wid = lax.axis_index("s") * 2 + lax.axis_index("c")
        base = wid * ppw
        lane16 = lax.iota(jnp.int32, 16)
        for j in range(ppw // 16):
            ident[pl.ds(j * 16, 16)] = base + j * 16 + lane16
        pltpu.sync_copy(i0_hbm.at[pl.ds(base, ppw)], i0)
        pltpu.sync_copy(i1_hbm.at[pl.ds(base, ppw)], i1)
        pltpu.sync_copy(i2_hbm.at[pl.ds(base, ppw)], i2)
        pltpu.sync_copy(w0_hbm.at[pl.ds(base, ppw)], w0v)
        pltpu.sync_copy(w1_hbm.at[pl.ds(base, ppw)], w1v)
        pltpu.sync_copy(w2_hbm.at[pl.ds(base, ppw)], w2v)
        bufs = ((r0a, r1a, r2a, ska, semA), (r0b, r1b, r2b, skb, semB))

        def fire(g, bs):
            r0, r1, r2, sk, sem = bs
            sl = pl.ds(g * _GP, _GP)
            return (
                pltpu.async_copy(f2_hbm.at[i0.at[sl]], r0, sem),
                pltpu.async_copy(f2_hbm.at[i1.at[sl]], r1, sem),
                pltpu.async_copy(f2_hbm.at[i2.at[sl]], r2, sem),
                pltpu.async_copy(skip_hbm.at[ident.at[sl]], sk, sem),
            )

        pend = fire(0, bufs[0])
        for g in range(ng):
            for c in pend:
                c.wait()
            r0, r1, r2, sk, _ = bufs[g % 2]
            if g + 1 < ng:
                pend = fire(g + 1, bufs[(g + 1) % 2])

            def body(p, carry, g=g, r0=r0, r1=r1, r2=r2, sk=sk):
                pg = g * _GP + p
                pv = jnp.full((16,), pg, jnp.int32)
                w0 = plsc.load_gather(w0v, [pv])
                w1 = plsc.load_gather(w1v, [pv])
                w2 = plsc.load_gather(w2v, [pv])
                for c in range(_COUT // 16):
                    sl = pl.ds(c * 16, 16)
                    fsl = pl.ds(pg * _COUT + c * 16, 16)
                    acc = (w0 * r0[p, sl] + w1 * r1[p, sl] + w2 * r2[p, sl]
                           + sk[p, sl])
                    out_v[fsl] = acc
                return carry

            lax.fori_loop(0, _GP, body, 0)
        pltpu.sync_copy(out_v, out_hbm.at[pl.ds(base * _COUT, ppw * _COUT)])

    return k(f2, idx3[0], idx3[1], idx3[2], w3[0], w3[1], w3[2], skip)


def kernel(feats, xyz, support_xyz, offset, support_offset, support_feats,
           ln1_g, ln1_b, W1, b1, ln2_g, ln2_b, W2, b2):
    cxyz8 = jnp.pad(xyz, ((0, 0), (0, 5)))
    sxyzT = jnp.pad(support_xyz.T[None], ((0, 0), (0, 5), (0, 0)))
    f2 = _f2_call(feats, ln2_g.reshape(1, _CIN), ln2_b.reshape(1, _CIN),
                  W2, b2.reshape(1, _COUT))
    halves = []
    for b0 in (0, 1, 2, 3):
        i0a, i1a, i2a, w0a, w1a, w2a, skip = _knn_call(
            sxyzT, cxyz8, support_feats, ln1_g.reshape(1, _COUT),
            ln1_b.reshape(1, _COUT), W1, b1.reshape(1, _COUT), b0, 1)
        halves.append(_sc_combine(f2, (i0a, i1a, i2a), (w0a, w1a, w2a),
                                  skip, _NF // 4))
    out = jnp.concatenate(halves)
    return (out.reshape(_NF, _COUT), support_xyz, support_offset)


# speedup vs baseline: 1.0028x; 1.0028x over previous
import functools

import jax
import jax.numpy as jnp
from jax import lax
from jax.experimental import pallas as pl
from jax.experimental.pallas import tpu as pltpu
from jax.experimental.pallas import tpu_sc as plsc

_B = 4
_NC = 4096
_NF = 16384
_NPB = _NC // _B
_MPB = _NF // _B
_CIN = 96
_COUT = 48
_MT = 512
_CT = 512

_NW = 32
_PPW = _NF // _NW
_GP = 64
_NG = _PPW // _GP


def _f2_body(x_ref, g_ref, b_ref, w_ref, bias_ref, o_ref):
    x = x_ref[...]
    mu = jnp.mean(x, axis=1, keepdims=True)
    xc = x - mu
    var = jnp.mean(xc * xc, axis=1, keepdims=True)
    y = xc / jnp.sqrt(var + 1e-5) * g_ref[...] + b_ref[...]
    f2 = (jnp.dot(y, w_ref[...], preferred_element_type=jnp.float32,
                  precision=lax.Precision.HIGHEST)
          + bias_ref[...])
    o_ref[...] = jnp.concatenate(
        [f2, jnp.zeros((_CT, 128 - _COUT), jnp.float32)], axis=1)


def _f2_call(feats, g, b, w, bias):
    return pl.pallas_call(
        _f2_body,
        grid=(_NC // _CT,),
        in_specs=[
            pl.BlockSpec((_CT, _CIN), lambda i: (i, 0)),
            pl.BlockSpec((1, _CIN), lambda i: (0, 0)),
            pl.BlockSpec((1, _CIN), lambda i: (0, 0)),
            pl.BlockSpec((_CIN, _COUT), lambda i: (0, 0)),
            pl.BlockSpec((1, _COUT), lambda i: (0, 0)),
        ],
        out_specs=pl.BlockSpec((_CT, 128), lambda i: (i, 0)),
        out_shape=jax.ShapeDtypeStruct((_NC, 128), jnp.float32),
    )(feats, g, b, w, bias)


def _knn_body(sxyz_ref, cxyz_ref, sf_ref, g_ref, bln_ref, w1_ref, b1_ref,
              i0_ref, i1_ref, i2_ref, w0_ref, w1o_ref, w2_ref, skip_ref,
              b0=0):
    b = b0 + pl.program_id(0)
    fx = sxyz_ref[0, 0:1, :]
    fy = sxyz_ref[0, 1:2, :]
    fz = sxyz_ref[0, 2:3, :]
    cx = cxyz_ref[:, 0:1]
    cy = cxyz_ref[:, 1:2]
    cz = cxyz_ref[:, 2:3]
    dx = cx - fx
    dy = cy - fy
    dz = cz - fz
    d2 = dx * dx + dy * dy + dz * dz
    iotaf = lax.broadcasted_iota(jnp.int32, (_NPB, _MT), 0).astype(jnp.float32)

    idxs = []
    ws = []
    for k in range(3):
        minv = jnp.min(d2, axis=0, keepdims=True)
        cand = jnp.where(d2 == minv, iotaf, jnp.float32(_NPB))
        aminf = jnp.min(cand, axis=0, keepdims=True)
        if k < 2:
            d2 = jnp.where(iotaf == aminf, jnp.float32(jnp.inf), d2)
        dist = jnp.sqrt(jnp.maximum(minv, 1e-12))
        ws.append(1.0 / (dist + 1e-8))
        idxs.append(aminf)
    wsum = ws[0] + ws[1] + ws[2]
    ws = [w / wsum for w in ws]

    gidx = [i.astype(jnp.int32) + b * _NPB for i in idxs]
    i0_ref[...] = gidx[0].reshape(_MT)
    i1_ref[...] = gidx[1].reshape(_MT)
    i2_ref[...] = gidx[2].reshape(_MT)
    w0_ref[...] = ws[0].reshape(_MT)
    w1o_ref[...] = ws[1].reshape(_MT)
    w2_ref[...] = ws[2].reshape(_MT)

    x = sf_ref[...]
    mu = jnp.mean(x, axis=1, keepdims=True)
    xc = x - mu
    var = jnp.mean(xc * xc, axis=1, keepdims=True)
    y = xc / jnp.sqrt(var + 1e-5) * g_ref[...] + bln_ref[...]
    sk = (jnp.dot(y, w1_ref[...], preferred_element_type=jnp.float32,
                  precision=lax.Precision.HIGHEST)
          + b1_ref[...])
    skip_ref[...] = jnp.concatenate(
        [sk, jnp.zeros((_MT, 128 - _COUT), jnp.float32)], axis=1)


def _knn_call(sxyzT, cxyz8, sfeat, g, bln, w1, b1, b0, nb):
    nt = _MPB // _MT
    nh = nb * _MPB
    body = functools.partial(_knn_body, b0=b0)
    return pl.pallas_call(
        body,
        grid=(nb, nt),
        in_specs=[
            pl.BlockSpec((1, 8, _MT), lambda b, t: (0, 0, (b0 + b) * nt + t)),
            pl.BlockSpec((_NPB, 8), lambda b, t: (b0 + b, 0)),
            pl.BlockSpec((_MT, _COUT), lambda b, t: ((b0 + b) * nt + t, 0)),
            pl.BlockSpec((1, _COUT), lambda b, t: (0, 0)),
            pl.BlockSpec((1, _COUT), lambda b, t: (0, 0)),
            pl.BlockSpec((_COUT, _COUT), lambda b, t: (0, 0)),
            pl.BlockSpec((1, _COUT), lambda b, t: (0, 0)),
        ],
        out_specs=[
            pl.BlockSpec((_MT,), lambda b, t: (b * nt + t,)),
            pl.BlockSpec((_MT,), lambda b, t: (b * nt + t,)),
            pl.BlockSpec((_MT,), lambda b, t: (b * nt + t,)),
            pl.BlockSpec((_MT,), lambda b, t: (b * nt + t,)),
            pl.BlockSpec((_MT,), lambda b, t: (b * nt + t,)),
            pl.BlockSpec((_MT,), lambda b, t: (b * nt + t,)),
            pl.BlockSpec((_MT, 128), lambda b, t: (b * nt + t, 0)),
        ],
        out_shape=[
            jax.ShapeDtypeStruct((nh,), jnp.int32),
            jax.ShapeDtypeStruct((nh,), jnp.int32),
            jax.ShapeDtypeStruct((nh,), jnp.int32),
            jax.ShapeDtypeStruct((nh,), jnp.float32),
            jax.ShapeDtypeStruct((nh,), jnp.float32),
            jax.ShapeDtypeStruct((nh,), jnp.float32),
            jax.ShapeDtypeStruct((nh, 128), jnp.float32),
        ],
    )(sxyzT, cxyz8, sfeat, g, bln, w1, b1)


def _sc_combine(f2, idx3, w3, skip, npts):
    ppw = npts // _NW
    ng = ppw // _GP
    mesh = plsc.VectorSubcoreMesh(core_axis_name="c", subcore_axis_name="s")

    @functools.partial(
        pl.kernel,
        out_type=jax.ShapeDtypeStruct((npts * _COUT,), jnp.float32),
        mesh=mesh,
        scratch_types=[
            pltpu.VMEM((ppw,), jnp.int32),
            pltpu.VMEM((ppw,), jnp.int32),
            pltpu.VMEM((ppw,), jnp.int32),
            pltpu.VMEM((ppw,), jnp.float32),
            pltpu.VMEM((ppw,), jnp.float32),
            pltpu.VMEM((ppw,), jnp.float32),
            pltpu.VMEM((ppw,), jnp.int32),
            pltpu.VMEM((ppw * _COUT,), jnp.float32),
            pltpu.VMEM((_GP, 128), jnp.float32),
            pltpu.VMEM((_GP, 128), jnp.float32),
            pltpu.VMEM((_GP, 128), jnp.float32),
            pltpu.VMEM((_GP, 128), jnp.float32),
            pltpu.VMEM((_GP, 128), jnp.float32),
            pltpu.VMEM((_GP, 128), jnp.float32),
            pltpu.VMEM((_GP, 128), jnp.float32),
            pltpu.VMEM((_GP, 128), jnp.float32),
            pltpu.SemaphoreType.DMA,
            pltpu.SemaphoreType.DMA,
        ],
        compiler_params=pltpu.CompilerParams(needs_layout_passes=False),
    )
    def k(f2_hbm, i0_hbm, i1_hbm, i2_hbm, w0_hbm, w1_hbm, w2_hbm,
          skip_hbm, out_hbm,
          i0, i1, i2, w0v, w1v, w2v, ident, out_v,
          r0a, r1a, r2a, ska, r0b, r1b, r2b, skb, semA, semB):
        wid = lax.axis_index("s") * 2 + lax.axis_index("c")
        base = wid * ppw
        lane16 = lax.iota(jnp.int32, 16)
        for j in range(ppw // 16):
            ident[pl.ds(j * 16, 16)] = base + j * 16 + lane16
        pltpu.sync_copy(i0_hbm.at[pl.ds(base, ppw)], i0)
        pltpu.sync_copy(i1_hbm.at[pl.ds(base, ppw)], i1)
        pltpu.sync_copy(i2_hbm.at[pl.ds(base, ppw)], i2)
        pltpu.sync_copy(w0_hbm.at[pl.ds(base, ppw)], w0v)
        pltpu.sync_copy(w1_hbm.at[pl.ds(base, ppw)], w1v)
        pltpu.sync_copy(w2_hbm.at[pl.ds(base, ppw)], w2v)
        bufs = ((r0a, r1a, r2a, ska, semA), (r0b, r1b, r2b, skb, semB))

        def fire(g, bs):
            r0, r1, r2, sk, sem = bs
            sl = pl.ds(g * _GP, _GP)
            return (
                pltpu.async_copy(f2_hbm.at[i0.at[sl]], r0, sem),
                pltpu.async_copy(f2_hbm.at[i1.at[sl]], r1, sem),
                pltpu.async_copy(f2_hbm.at[i2.at[sl]], r2, sem),
                pltpu.async_copy(skip_hbm.at[ident.at[sl]], sk, sem),
            )

        pend = fire(0, bufs[0])
        for g in range(ng):
            for c in pend:
                c.wait()
            r0, r1, r2, sk, _ = bufs[g % 2]
            if g + 1 < ng:
                pend = fire(g + 1, bufs[(g + 1) % 2])

            def body(p, carry, g=g, r0=r0, r1=r1, r2=r2, sk=sk):
                pg = g * _GP + p
                pv = jnp.full((16,), pg, jnp.int32)
                w0 = plsc.load_gather(w0v, [pv])
                w1 = plsc.load_gather(w1v, [pv])
                w2 = plsc.load_gather(w2v, [pv])
                for c in range(_COUT // 16):
                    sl = pl.ds(c * 16, 16)
                    fsl = pl.ds(pg * _COUT + c * 16, 16)
                    acc = (w0 * r0[p, sl] + w1 * r1[p, sl] + w2 * r2[p, sl]
                           + sk[p, sl])
                    out_v[fsl] = acc
                return carry

            lax.fori_loop(0, _GP, body, 0)
        pltpu.sync_copy(out_v, out_hbm.at[pl.ds(base * _COUT, ppw * _COUT)])

    return k(f2, idx3[0], idx3[1], idx3[2], w3[0], w3[1], w3[2], skip)


def kernel(feats, xyz, support_xyz, offset, support_offset, support_feats,
           ln1_g, ln1_b, W1, b1, ln2_g, ln2_b, W2, b2):
    cxyz8 = jnp.pad(xyz, ((0, 0), (0, 5)))
    sxyzT = jnp.pad(support_xyz.T[None], ((0, 0), (0, 5), (0, 0)))
    f2 = _f2_call(feats, ln2_g.reshape(1, _CIN), ln2_b.reshape(1, _CIN),
                  W2, b2.reshape(1, _COUT))
    halves = []
    for b0 in (0, 1, 2, 3):
        i0a, i1a, i2a, w0a, w1a, w2a, skip = _knn_call(
            sxyzT, cxyz8, support_feats, ln1_g.reshape(1, _COUT),
            ln1_b.reshape(1, _COUT), W1, b1.reshape(1, _COUT), b0, 1)
        halves.append(_sc_combine(f2, (i0a, i1a, i2a), (w0a, w1a, w2a),
                                  skip, _NF // 4))
    out = jnp.concatenate(halves)
    return (out.reshape(_NF, _COUT), support_xyz, support_offset)
